# Initial kernel scaffold; baseline (speedup 1.0000x reference)
import jax
import jax.numpy as jnp
from jax.experimental import pallas as pl


def kernel(x, edge_index, batch, W1, b1, g1, beta1, W2, b2, g2, beta2, Wl, bl):
    raise NotImplementedError("write your pallas kernel here")


# speedup vs baseline: 7.0245x; 7.0245x over previous
import jax
import jax.numpy as jnp
from jax import lax
from jax.experimental import pallas as pl
from jax.experimental.pallas import tpu as pltpu
from jax.experimental.pallas import tpu_sc as plsc

N = 10000
E = 640000
F_IN = 9
H = 128
G = 128

NC = 2
NS = 16
NW = NC * NS
CHUNK = 128
KB = 8
KO = 20
K = KB * KO
EPT = K * CHUNK
E_PAD = NW * EPT
RPT = 632
N_ACC = NS * RPT

_MESH = plsc.VectorSubcoreMesh(
    core_axis_name="c", subcore_axis_name="s", num_cores=NC, num_subcores=NS)


def _make_agg(W):
  def body(y_hbm, src_hbm, dst_hbm, z_hbm, out_hbm,
           acc, idx_s, idx_d, rows_v, sem):
    c = lax.axis_index("c")
    s = lax.axis_index("s")
    w = c * NS + s
    pltpu.sync_copy(z_hbm, acc.at[pl.ds(s * RPT, RPT)])
    plsc.subcore_barrier()

    def body_outer(ko, carry):
      pltpu.sync_copy(src_hbm.at[w, pl.ds(ko * KB, KB)], idx_s)
      pltpu.sync_copy(dst_hbm.at[w, pl.ds(ko * KB, KB)], idx_d)

      def inner(j, carry2):
        pltpu.async_copy(y_hbm.at[idx_s.at[j]], rows_v, sem).wait()
        pltpu.sync_copy(rows_v, acc.at[idx_d.at[j]], add=True)
        return carry2

      lax.fori_loop(0, KB, inner, 0, unroll=False)
      return carry

    lax.fori_loop(0, KO, body_outer, 0, unroll=False)
    plsc.subcore_barrier()
    pltpu.sync_copy(acc.at[pl.ds(s * RPT, RPT)],
                    out_hbm.at[c, pl.ds(s * RPT, RPT)])

  return pl.kernel(
      body,
      out_type=jax.ShapeDtypeStruct((NC, N_ACC, W), jnp.float32),
      mesh=_MESH,
      scratch_types=[
          pltpu.VMEM_SHARED((N_ACC, W), jnp.float32),
          pltpu.VMEM((KB, CHUNK), jnp.int32),
          pltpu.VMEM((KB, CHUNK), jnp.int32),
          pltpu.VMEM((CHUNK, W), jnp.float32),
          pltpu.SemaphoreType.DMA,
      ],
  )


_agg128 = _make_agg(H)


def _degc_body(ones_hbm, dst_hbm, z_hbm, out_hbm, acc, idx_d, rows_v):
  c = lax.axis_index("c")
  s = lax.axis_index("s")
  w = c * NS + s
  pltpu.sync_copy(ones_hbm, rows_v)
  pltpu.sync_copy(z_hbm, acc.at[pl.ds(s * RPT, RPT)])
  plsc.subcore_barrier()

  def body_outer(ko, carry):
    pltpu.sync_copy(dst_hbm.at[w, pl.ds(ko * KB, KB)], idx_d)

    def inner(j, carry2):
      pltpu.sync_copy(rows_v, acc.at[idx_d.at[j]], add=True)
      return carry2

    lax.fori_loop(0, KB, inner, 0, unroll=False)
    return carry

  lax.fori_loop(0, KO, body_outer, 0, unroll=False)
  plsc.subcore_barrier()
  pltpu.sync_copy(acc.at[pl.ds(s * RPT, RPT)],
                  out_hbm.at[c, pl.ds(s * RPT, RPT)])


_deg_call = pl.kernel(
    _degc_body,
    out_type=jax.ShapeDtypeStruct((NC, N_ACC, H), jnp.float32),
    mesh=_MESH,
    scratch_types=[
        pltpu.VMEM_SHARED((N_ACC, H), jnp.float32),
        pltpu.VMEM((KB, CHUNK), jnp.int32),
        pltpu.VMEM((CHUNK, H), jnp.float32),
    ],
)


def _prep1_body(x_ref, w1_ref, dega_ref, degb_ref, y1_ref, dinv_ref):
  deg = dega_ref[...] + degb_ref[...] + 1.0
  dinv = lax.rsqrt(deg)
  xw = jnp.dot(x_ref[...], w1_ref[...], preferred_element_type=jnp.float32)
  dinv_ref[...] = dinv
  y1_ref[...] = xw * dinv


def _mid_body(a_ref, b_ref, y_ref, dinv_ref, b1_ref, g1_ref, bt1_ref,
              w2_ref, y2_ref):
  dinv = dinv_ref[...]
  pre = dinv * (a_ref[...] + b_ref[...] + y_ref[...]) + b1_ref[...]
  mu = jnp.mean(pre, axis=0, keepdims=True)
  d = pre - mu
  var = jnp.mean(d * d, axis=0, keepdims=True)
  h = jnp.maximum(g1_ref[...] * d * lax.rsqrt(var + 1e-5) + bt1_ref[...], 0.0)
  y2_ref[...] = jnp.dot(h, w2_ref[...],
                        preferred_element_type=jnp.float32) * dinv


def _final_body(a_ref, b_ref, y_ref, dinv_ref, b2_ref, g2_ref, bt2_ref,
                batch_ref, wl_ref, bl_ref, out_ref, pooled_ref):
  dinv = dinv_ref[...]
  pre = dinv * (a_ref[...] + b_ref[...] + y_ref[...]) + b2_ref[...]
  mu = jnp.mean(pre, axis=0, keepdims=True)
  d = pre - mu
  var = jnp.mean(d * d, axis=0, keepdims=True)
  h = jnp.maximum(g2_ref[...] * d * lax.rsqrt(var + 1e-5) + bt2_ref[...], 0.0)
  bt = batch_ref[...]

  def body(g, carry):
    vals = jnp.where(bt == g, h, -1.0)
    pooled_ref[pl.ds(g, 1), :] = jnp.max(vals, axis=0, keepdims=True)
    return carry

  lax.fori_loop(0, G, body, 0, unroll=False)
  pooled = pooled_ref[...]
  pooled = jnp.where(pooled >= 0.0, pooled, 0.0)
  logits = jnp.dot(pooled, wl_ref[...], preferred_element_type=jnp.float32)
  out_ref[...] = 1.0 / (1.0 + jnp.exp(-(logits + bl_ref[...])))


def _tc_call(body, out_shapes, n_in, scratch_shapes=()):
  return pl.pallas_call(
      body,
      out_shape=out_shapes,
      in_specs=[pl.BlockSpec(memory_space=pltpu.VMEM) for _ in range(n_in)],
      out_specs=[pl.BlockSpec(memory_space=pltpu.VMEM) for _ in out_shapes],
      scratch_shapes=list(scratch_shapes),
  )


@jax.jit
def kernel(x, edge_index, batch, W1, b1, g1, beta1, W2, b2, g2, beta2, Wl, bl):
  src = edge_index[0]
  dst = edge_index[1]
  pad = E_PAD - E
  src_p = jnp.concatenate([src, jnp.zeros((pad,), jnp.int32)]).reshape(NW, K, CHUNK)
  dst_p = jnp.concatenate([dst, jnp.full((pad,), N, jnp.int32)]).reshape(NW, K, CHUNK)

  z128 = jnp.zeros((RPT, H), jnp.float32)

  degp = _deg_call(jnp.ones((CHUNK, H), jnp.float32), dst_p, z128)
  dega = degp[0, :N, 0:1]
  degb = degp[1, :N, 0:1]

  x_pad = jnp.pad(x, ((0, 0), (0, 16 - F_IN)))
  w1_pad = jnp.pad(W1, ((0, 16 - F_IN), (0, 0)))

  y1, dinv = _tc_call(
      _prep1_body,
      [jax.ShapeDtypeStruct((N, H), jnp.float32),
       jax.ShapeDtypeStruct((N, 1), jnp.float32)],
      4)(x_pad, w1_pad, dega, degb)

  agg1 = _agg128(y1, src_p, dst_p, z128)

  y2 = _tc_call(
      _mid_body,
      [jax.ShapeDtypeStruct((N, H), jnp.float32)],
      8)(agg1[0, :N], agg1[1, :N], y1, dinv,
         b1.reshape(1, H), g1.reshape(1, H), beta1.reshape(1, H), W2)[0]

  agg2 = _agg128(y2, src_p, dst_p, z128)

  out = _tc_call(
      _final_body,
      [jax.ShapeDtypeStruct((G, 1), jnp.float32)],
      10,
      scratch_shapes=[pltpu.VMEM((G, H), jnp.float32)],
  )(agg2[0, :N], agg2[1, :N], y2, dinv,
    b2.reshape(1, H), g2.reshape(1, H), beta2.reshape(1, H),
    batch.reshape(N, 1), Wl, bl.reshape(1, 1))[0]
  return out


# speedup vs baseline: 7.4568x; 1.0615x over previous
import jax
import jax.numpy as jnp
from jax import lax
from jax.experimental import pallas as pl
from jax.experimental.pallas import tpu as pltpu
from jax.experimental.pallas import tpu_sc as plsc

N = 10000
E = 640000
F_IN = 9
H = 128
G = 128

NC = 2
NS = 16
NW = NC * NS
CHUNK = 128
KB = 8
KO = 20
K = KB * KO
EPT = K * CHUNK
E_PAD = NW * EPT
RPT = 632
N_ACC = NS * RPT

_MESH = plsc.VectorSubcoreMesh(
    core_axis_name="c", subcore_axis_name="s", num_cores=NC, num_subcores=NS)


def _make_agg(W):
  def body(y_hbm, src_hbm, dst_hbm, z_hbm, out_hbm,
           acc, idx_s, idx_d, rows0, rows1, sg0, sg1, ss0, ss1):
    c = lax.axis_index("c")
    s = lax.axis_index("s")
    w = c * NS + s
    rows = [rows0, rows1]
    sg = [sg0, sg1]
    ss = [ss0, ss1]
    pltpu.sync_copy(z_hbm, acc.at[pl.ds(s * RPT, RPT)])
    plsc.subcore_barrier()

    def body_outer(ko, carry):
      pltpu.sync_copy(src_hbm.at[w, pl.ds(ko * KB, KB)], idx_s)
      pltpu.sync_copy(dst_hbm.at[w, pl.ds(ko * KB, KB)], idx_d)
      g = [None] * KB
      sc = [None] * KB
      g[0] = pltpu.async_copy(y_hbm.at[idx_s.at[0]], rows[0], sg[0])
      for j in range(KB):
        b = j % 2
        g[j].wait()
        if j + 1 < KB:
          if j >= 1:
            sc[j - 1].wait()
          g[j + 1] = pltpu.async_copy(
              y_hbm.at[idx_s.at[j + 1]], rows[(j + 1) % 2], sg[(j + 1) % 2])
        sc[j] = pltpu.async_copy(rows[b], acc.at[idx_d.at[j]], ss[b],
                                 add=True)
      sc[KB - 2].wait()
      sc[KB - 1].wait()
      return carry

    lax.fori_loop(0, KO, body_outer, 0, unroll=False)
    plsc.subcore_barrier()
    pltpu.sync_copy(acc.at[pl.ds(s * RPT, RPT)],
                    out_hbm.at[c, pl.ds(s * RPT, RPT)])

  return pl.kernel(
      body,
      out_type=jax.ShapeDtypeStruct((NC, N_ACC, W), jnp.float32),
      mesh=_MESH,
      scratch_types=[
          pltpu.VMEM_SHARED((N_ACC, W), jnp.float32),
          pltpu.VMEM((KB, CHUNK), jnp.int32),
          pltpu.VMEM((KB, CHUNK), jnp.int32),
          pltpu.VMEM((CHUNK, W), jnp.float32),
          pltpu.VMEM((CHUNK, W), jnp.float32),
          pltpu.SemaphoreType.DMA,
          pltpu.SemaphoreType.DMA,
          pltpu.SemaphoreType.DMA,
          pltpu.SemaphoreType.DMA,
      ],
  )


_agg128 = _make_agg(H)


def _degc_body(ones_hbm, dst_hbm, z_hbm, out_hbm, acc, idx_d, rows_v,
               ss0, ss1):
  c = lax.axis_index("c")
  s = lax.axis_index("s")
  w = c * NS + s
  ss = [ss0, ss1]
  pltpu.sync_copy(ones_hbm, rows_v)
  pltpu.sync_copy(z_hbm, acc.at[pl.ds(s * RPT, RPT)])
  plsc.subcore_barrier()

  def body_outer(ko, carry):
    pltpu.sync_copy(dst_hbm.at[w, pl.ds(ko * KB, KB)], idx_d)
    sc = [None] * KB
    for j in range(KB):
      b = j % 2
      if j >= 2:
        sc[j - 2].wait()
      sc[j] = pltpu.async_copy(rows_v, acc.at[idx_d.at[j]], ss[b], add=True)
    sc[KB - 2].wait()
    sc[KB - 1].wait()
    return carry

  lax.fori_loop(0, KO, body_outer, 0, unroll=False)
  plsc.subcore_barrier()
  pltpu.sync_copy(acc.at[pl.ds(s * RPT, RPT)],
                  out_hbm.at[c, pl.ds(s * RPT, RPT)])


_deg_call = pl.kernel(
    _degc_body,
    out_type=jax.ShapeDtypeStruct((NC, N_ACC, H), jnp.float32),
    mesh=_MESH,
    scratch_types=[
        pltpu.VMEM_SHARED((N_ACC, H), jnp.float32),
        pltpu.VMEM((KB, CHUNK), jnp.int32),
        pltpu.VMEM((CHUNK, H), jnp.float32),
        pltpu.SemaphoreType.DMA,
        pltpu.SemaphoreType.DMA,
    ],
)


def _prep1_body(x_ref, w1_ref, dega_ref, degb_ref, y1_ref, dinv_ref):
  deg = dega_ref[...] + degb_ref[...] + 1.0
  dinv = lax.rsqrt(deg)
  xw = jnp.dot(x_ref[...], w1_ref[...], preferred_element_type=jnp.float32)
  dinv_ref[...] = dinv
  y1_ref[...] = xw * dinv


def _mid_body(a_ref, b_ref, y_ref, dinv_ref, b1_ref, g1_ref, bt1_ref,
              w2_ref, y2_ref):
  dinv = dinv_ref[...]
  pre = dinv * (a_ref[...] + b_ref[...] + y_ref[...]) + b1_ref[...]
  mu = jnp.mean(pre, axis=0, keepdims=True)
  d = pre - mu
  var = jnp.mean(d * d, axis=0, keepdims=True)
  h = jnp.maximum(g1_ref[...] * d * lax.rsqrt(var + 1e-5) + bt1_ref[...], 0.0)
  y2_ref[...] = jnp.dot(h, w2_ref[...],
                        preferred_element_type=jnp.float32) * dinv


def _final_body(a_ref, b_ref, y_ref, dinv_ref, b2_ref, g2_ref, bt2_ref,
                batch_ref, wl_ref, bl_ref, out_ref, pooled_ref):
  dinv = dinv_ref[...]
  pre = dinv * (a_ref[...] + b_ref[...] + y_ref[...]) + b2_ref[...]
  mu = jnp.mean(pre, axis=0, keepdims=True)
  d = pre - mu
  var = jnp.mean(d * d, axis=0, keepdims=True)
  h = jnp.maximum(g2_ref[...] * d * lax.rsqrt(var + 1e-5) + bt2_ref[...], 0.0)
  bt = batch_ref[...]

  def body(g, carry):
    vals = jnp.where(bt == g, h, -1.0)
    pooled_ref[pl.ds(g, 1), :] = jnp.max(vals, axis=0, keepdims=True)
    return carry

  lax.fori_loop(0, G, body, 0, unroll=False)
  pooled = pooled_ref[...]
  pooled = jnp.where(pooled >= 0.0, pooled, 0.0)
  logits = jnp.dot(pooled, wl_ref[...], preferred_element_type=jnp.float32)
  out_ref[...] = 1.0 / (1.0 + jnp.exp(-(logits + bl_ref[...])))


def _tc_call(body, out_shapes, n_in, scratch_shapes=()):
  return pl.pallas_call(
      body,
      out_shape=out_shapes,
      in_specs=[pl.BlockSpec(memory_space=pltpu.VMEM) for _ in range(n_in)],
      out_specs=[pl.BlockSpec(memory_space=pltpu.VMEM) for _ in out_shapes],
      scratch_shapes=list(scratch_shapes),
  )


@jax.jit
def kernel(x, edge_index, batch, W1, b1, g1, beta1, W2, b2, g2, beta2, Wl, bl):
  src = edge_index[0]
  dst = edge_index[1]
  pad = E_PAD - E
  src_p = jnp.concatenate([src, jnp.zeros((pad,), jnp.int32)]).reshape(NW, K, CHUNK)
  dst_p = jnp.concatenate([dst, jnp.full((pad,), N, jnp.int32)]).reshape(NW, K, CHUNK)

  z128 = jnp.zeros((RPT, H), jnp.float32)

  degp = _deg_call(jnp.ones((CHUNK, H), jnp.float32), dst_p, z128)
  dega = degp[0, :N, 0:1]
  degb = degp[1, :N, 0:1]

  x_pad = jnp.pad(x, ((0, 0), (0, 16 - F_IN)))
  w1_pad = jnp.pad(W1, ((0, 16 - F_IN), (0, 0)))

  y1, dinv = _tc_call(
      _prep1_body,
      [jax.ShapeDtypeStruct((N, H), jnp.float32),
       jax.ShapeDtypeStruct((N, 1), jnp.float32)],
      4)(x_pad, w1_pad, dega, degb)

  agg1 = _agg128(y1, src_p, dst_p, z128)

  y2 = _tc_call(
      _mid_body,
      [jax.ShapeDtypeStruct((N, H), jnp.float32)],
      8)(agg1[0, :N], agg1[1, :N], y1, dinv,
         b1.reshape(1, H), g1.reshape(1, H), beta1.reshape(1, H), W2)[0]

  agg2 = _agg128(y2, src_p, dst_p, z128)

  out = _tc_call(
      _final_body,
      [jax.ShapeDtypeStruct((G, 1), jnp.float32)],
      10,
      scratch_shapes=[pltpu.VMEM((G, H), jnp.float32)],
  )(agg2[0, :N], agg2[1, :N], y2, dinv,
    b2.reshape(1, H), g2.reshape(1, H), beta2.reshape(1, H),
    batch.reshape(N, 1), Wl, bl.reshape(1, 1))[0]
  return out


# speedup vs baseline: 9.4052x; 1.2613x over previous
import jax
import jax.numpy as jnp
from jax import lax
from jax.experimental import pallas as pl
from jax.experimental.pallas import tpu as pltpu
from jax.experimental.pallas import tpu_sc as plsc

N = 10000
E = 640000
F_IN = 9
H = 128
G = 128

NC = 2
NS = 16
NW = NC * NS
CHUNK = 128
KB = 8
KO = 20
K = KB * KO
EPT = K * CHUNK
E_PAD = NW * EPT
TOT_CHUNKS = E_PAD // CHUNK
K0 = 256
K1 = 64
KO0 = K0 // KB
KO1 = K1 // KB
RPT = 632
N_ACC = NS * RPT

_MESH = plsc.VectorSubcoreMesh(
    core_axis_name="c", subcore_axis_name="s", num_cores=NC, num_subcores=NS)


def _make_agg(W):
  def body(y_hbm, src_hbm, dst_hbm, z_hbm, out_hbm,
           acc, idx_s, idx_d, rows0, rows1, sg0, sg1, ss0, ss1):
    c = lax.axis_index("c")
    s = lax.axis_index("s")
    base = jnp.where(c == 0, s * K0, NS * K0 + s * K1)
    n_blocks = jnp.where(c == 0, KO0, KO1)
    rows = [rows0, rows1]
    sg = [sg0, sg1]
    ss = [ss0, ss1]
    pltpu.sync_copy(z_hbm, acc.at[pl.ds(s * RPT, RPT)])
    plsc.subcore_barrier()

    def body_outer(ko, carry):
      off = base + ko * KB
      pltpu.sync_copy(src_hbm.at[pl.ds(off, KB)], idx_s)
      pltpu.sync_copy(dst_hbm.at[pl.ds(off, KB)], idx_d)
      g = [None] * KB
      sc = [None] * KB
      g[0] = pltpu.async_copy(y_hbm.at[idx_s.at[0]], rows[0], sg[0])
      for j in range(KB):
        b = j % 2
        g[j].wait()
        if j + 1 < KB:
          if j >= 1:
            sc[j - 1].wait()
          g[j + 1] = pltpu.async_copy(
              y_hbm.at[idx_s.at[j + 1]], rows[(j + 1) % 2], sg[(j + 1) % 2])
        sc[j] = pltpu.async_copy(rows[b], acc.at[idx_d.at[j]], ss[b],
                                 add=True)
      sc[KB - 2].wait()
      sc[KB - 1].wait()
      return carry

    lax.fori_loop(0, n_blocks, body_outer, 0, unroll=False)
    plsc.subcore_barrier()
    pltpu.sync_copy(acc.at[pl.ds(s * RPT, RPT)],
                    out_hbm.at[c, pl.ds(s * RPT, RPT)])

  return pl.kernel(
      body,
      out_type=jax.ShapeDtypeStruct((NC, N_ACC, W), jnp.float32),
      mesh=_MESH,
      scratch_types=[
          pltpu.VMEM_SHARED((N_ACC, W), jnp.float32),
          pltpu.VMEM((KB, CHUNK), jnp.int32),
          pltpu.VMEM((KB, CHUNK), jnp.int32),
          pltpu.VMEM((CHUNK, W), jnp.float32),
          pltpu.VMEM((CHUNK, W), jnp.float32),
          pltpu.SemaphoreType.DMA,
          pltpu.SemaphoreType.DMA,
          pltpu.SemaphoreType.DMA,
          pltpu.SemaphoreType.DMA,
      ],
  )


_agg128 = _make_agg(H)


def _degc_body(ones_hbm, dst_hbm, z_hbm, out_hbm, acc, idx_d, rows_v,
               ss0, ss1):
  c = lax.axis_index("c")
  s = lax.axis_index("s")
  w = c * NS + s
  ss = [ss0, ss1]
  pltpu.sync_copy(ones_hbm, rows_v)
  pltpu.sync_copy(z_hbm, acc.at[pl.ds(s * RPT, RPT)])
  plsc.subcore_barrier()

  def body_outer(ko, carry):
    pltpu.sync_copy(dst_hbm.at[pl.ds(w * K + ko * KB, KB)], idx_d)
    sc = [None] * KB
    for j in range(KB):
      b = j % 2
      if j >= 2:
        sc[j - 2].wait()
      sc[j] = pltpu.async_copy(rows_v, acc.at[idx_d.at[j]], ss[b], add=True)
    sc[KB - 2].wait()
    sc[KB - 1].wait()
    return carry

  lax.fori_loop(0, KO, body_outer, 0, unroll=False)
  plsc.subcore_barrier()
  pltpu.sync_copy(acc.at[pl.ds(s * RPT, RPT)],
                  out_hbm.at[c, pl.ds(s * RPT, RPT)])


_deg_call = pl.kernel(
    _degc_body,
    out_type=jax.ShapeDtypeStruct((NC, N_ACC, H), jnp.float32),
    mesh=_MESH,
    scratch_types=[
        pltpu.VMEM_SHARED((N_ACC, H), jnp.float32),
        pltpu.VMEM((KB, CHUNK), jnp.int32),
        pltpu.VMEM((CHUNK, H), jnp.float32),
        pltpu.SemaphoreType.DMA,
        pltpu.SemaphoreType.DMA,
    ],
)


def _prep1_body(x_ref, w1_ref, dega_ref, degb_ref, y1_ref, dinv_ref):
  deg = dega_ref[...] + degb_ref[...] + 1.0
  dinv = lax.rsqrt(deg)
  xw = jnp.dot(x_ref[...], w1_ref[...], preferred_element_type=jnp.float32)
  dinv_ref[...] = dinv
  y1_ref[...] = xw * dinv


def _mid_body(a_ref, b_ref, y_ref, dinv_ref, b1_ref, g1_ref, bt1_ref,
              w2_ref, y2_ref):
  dinv = dinv_ref[...]
  pre = dinv * (a_ref[...] + b_ref[...] + y_ref[...]) + b1_ref[...]
  mu = jnp.mean(pre, axis=0, keepdims=True)
  d = pre - mu
  var = jnp.mean(d * d, axis=0, keepdims=True)
  h = jnp.maximum(g1_ref[...] * d * lax.rsqrt(var + 1e-5) + bt1_ref[...], 0.0)
  y2_ref[...] = jnp.dot(h, w2_ref[...],
                        preferred_element_type=jnp.float32) * dinv


def _final_body(a_ref, b_ref, y_ref, dinv_ref, b2_ref, g2_ref, bt2_ref,
                batch_ref, wl_ref, bl_ref, out_ref, pooled_ref):
  dinv = dinv_ref[...]
  pre = dinv * (a_ref[...] + b_ref[...] + y_ref[...]) + b2_ref[...]
  mu = jnp.mean(pre, axis=0, keepdims=True)
  d = pre - mu
  var = jnp.mean(d * d, axis=0, keepdims=True)
  h = jnp.maximum(g2_ref[...] * d * lax.rsqrt(var + 1e-5) + bt2_ref[...], 0.0)
  bt = batch_ref[...]

  def body(g, carry):
    vals = jnp.where(bt == g, h, -1.0)
    pooled_ref[pl.ds(g, 1), :] = jnp.max(vals, axis=0, keepdims=True)
    return carry

  lax.fori_loop(0, G, body, 0, unroll=False)
  pooled = pooled_ref[...]
  pooled = jnp.where(pooled >= 0.0, pooled, 0.0)
  logits = jnp.dot(pooled, wl_ref[...], preferred_element_type=jnp.float32)
  out_ref[...] = 1.0 / (1.0 + jnp.exp(-(logits + bl_ref[...])))


def _tc_call(body, out_shapes, n_in, scratch_shapes=()):
  return pl.pallas_call(
      body,
      out_shape=out_shapes,
      in_specs=[pl.BlockSpec(memory_space=pltpu.VMEM) for _ in range(n_in)],
      out_specs=[pl.BlockSpec(memory_space=pltpu.VMEM) for _ in out_shapes],
      scratch_shapes=list(scratch_shapes),
  )


@jax.jit
def kernel(x, edge_index, batch, W1, b1, g1, beta1, W2, b2, g2, beta2, Wl, bl):
  src = edge_index[0]
  dst = edge_index[1]
  pad = E_PAD - E
  src_p = jnp.concatenate([src, jnp.zeros((pad,), jnp.int32)]).reshape(TOT_CHUNKS, CHUNK)
  dst_p = jnp.concatenate([dst, jnp.full((pad,), N, jnp.int32)]).reshape(TOT_CHUNKS, CHUNK)

  z128 = jnp.zeros((RPT, H), jnp.float32)

  degp = _deg_call(jnp.ones((CHUNK, H), jnp.float32), dst_p, z128)
  dega = degp[0, :N, 0:1]
  degb = degp[1, :N, 0:1]

  x_pad = jnp.pad(x, ((0, 0), (0, 16 - F_IN)))
  w1_pad = jnp.pad(W1, ((0, 16 - F_IN), (0, 0)))

  y1, dinv = _tc_call(
      _prep1_body,
      [jax.ShapeDtypeStruct((N, H), jnp.float32),
       jax.ShapeDtypeStruct((N, 1), jnp.float32)],
      4)(x_pad, w1_pad, dega, degb)

  agg1 = _agg128(y1, src_p, dst_p, z128)

  y2 = _tc_call(
      _mid_body,
      [jax.ShapeDtypeStruct((N, H), jnp.float32)],
      8)(agg1[0, :N], agg1[1, :N], y1, dinv,
         b1.reshape(1, H), g1.reshape(1, H), beta1.reshape(1, H), W2)[0]

  agg2 = _agg128(y2, src_p, dst_p, z128)

  out = _tc_call(
      _final_body,
      [jax.ShapeDtypeStruct((G, 1), jnp.float32)],
      10,
      scratch_shapes=[pltpu.VMEM((G, H), jnp.float32)],
  )(agg2[0, :N], agg2[1, :N], y2, dinv,
    b2.reshape(1, H), g2.reshape(1, H), beta2.reshape(1, H),
    batch.reshape(N, 1), Wl, bl.reshape(1, 1))[0]
  return out


# speedup vs baseline: 10.4119x; 1.1070x over previous
import jax
import jax.numpy as jnp
from jax import lax
from jax.experimental import pallas as pl
from jax.experimental.pallas import tpu as pltpu
from jax.experimental.pallas import tpu_sc as plsc

N = 10000
E = 640000
F_IN = 9
H = 128
G = 128

NC = 2
NS = 16
NW = NC * NS
CHUNK = 128
KB = 8
KO = 20
K = KB * KO
EPT = K * CHUNK
E_PAD = NW * EPT
TOT_CHUNKS = E_PAD // CHUNK
K0 = 280
K1 = 40
KO0 = K0 // KB
KO1 = K1 // KB
RPT = 632
N_ACC = NS * RPT

_MESH = plsc.VectorSubcoreMesh(
    core_axis_name="c", subcore_axis_name="s", num_cores=NC, num_subcores=NS)


def _make_agg(W):
  def body(y_hbm, src_hbm, dst_hbm, z_hbm, out_hbm,
           acc, idx_s, idx_d, rows0, rows1, sg0, sg1, ss0, ss1):
    c = lax.axis_index("c")
    s = lax.axis_index("s")
    base = jnp.where(c == 0, s * K0, NS * K0 + s * K1)
    n_blocks = jnp.where(c == 0, KO0, KO1)
    rows = [rows0, rows1]
    sg = [sg0, sg1]
    ss = [ss0, ss1]
    pltpu.sync_copy(z_hbm, acc.at[pl.ds(s * RPT, RPT)])
    plsc.subcore_barrier()

    def body_outer(ko, carry):
      off = base + ko * KB
      pltpu.sync_copy(src_hbm.at[pl.ds(off, KB)], idx_s)
      pltpu.sync_copy(dst_hbm.at[pl.ds(off, KB)], idx_d)
      g = [None] * KB
      sc = [None] * KB
      g[0] = pltpu.async_copy(y_hbm.at[idx_s.at[0]], rows[0], sg[0])
      for j in range(KB):
        b = j % 2
        g[j].wait()
        if j + 1 < KB:
          if j >= 1:
            sc[j - 1].wait()
          g[j + 1] = pltpu.async_copy(
              y_hbm.at[idx_s.at[j + 1]], rows[(j + 1) % 2], sg[(j + 1) % 2])
        sc[j] = pltpu.async_copy(rows[b], acc.at[idx_d.at[j]], ss[b],
                                 add=True)
      sc[KB - 2].wait()
      sc[KB - 1].wait()
      return carry

    lax.fori_loop(0, n_blocks, body_outer, 0, unroll=False)
    plsc.subcore_barrier()
    pltpu.sync_copy(acc.at[pl.ds(s * RPT, RPT)],
                    out_hbm.at[c, pl.ds(s * RPT, RPT)])

  return pl.kernel(
      body,
      out_type=jax.ShapeDtypeStruct((NC, N_ACC, W), jnp.float32),
      mesh=_MESH,
      scratch_types=[
          pltpu.VMEM_SHARED((N_ACC, W), jnp.float32),
          pltpu.VMEM((KB, CHUNK), jnp.int32),
          pltpu.VMEM((KB, CHUNK), jnp.int32),
          pltpu.VMEM((CHUNK, W), jnp.float32),
          pltpu.VMEM((CHUNK, W), jnp.float32),
          pltpu.SemaphoreType.DMA,
          pltpu.SemaphoreType.DMA,
          pltpu.SemaphoreType.DMA,
          pltpu.SemaphoreType.DMA,
      ],
  )


_agg128 = _make_agg(H)


def _degc_body(ones_hbm, dst_hbm, z_hbm, out_hbm, acc, idx_d, rows_v,
               ss0, ss1):
  c = lax.axis_index("c")
  s = lax.axis_index("s")
  w = c * NS + s
  ss = [ss0, ss1]
  pltpu.sync_copy(ones_hbm, rows_v)
  pltpu.sync_copy(z_hbm, acc.at[pl.ds(s * RPT, RPT)])
  plsc.subcore_barrier()

  def body_outer(ko, carry):
    pltpu.sync_copy(dst_hbm.at[pl.ds(w * K + ko * KB, KB)], idx_d)
    sc = [None] * KB
    for j in range(KB):
      b = j % 2
      if j >= 2:
        sc[j - 2].wait()
      sc[j] = pltpu.async_copy(rows_v, acc.at[idx_d.at[j]], ss[b], add=True)
    sc[KB - 2].wait()
    sc[KB - 1].wait()
    return carry

  lax.fori_loop(0, KO, body_outer, 0, unroll=False)
  plsc.subcore_barrier()
  pltpu.sync_copy(acc.at[pl.ds(s * RPT, RPT)],
                  out_hbm.at[c, pl.ds(s * RPT, RPT)])


_deg_call = pl.kernel(
    _degc_body,
    out_type=jax.ShapeDtypeStruct((NC, N_ACC, H), jnp.float32),
    mesh=_MESH,
    scratch_types=[
        pltpu.VMEM_SHARED((N_ACC, H), jnp.float32),
        pltpu.VMEM((KB, CHUNK), jnp.int32),
        pltpu.VMEM((CHUNK, H), jnp.float32),
        pltpu.SemaphoreType.DMA,
        pltpu.SemaphoreType.DMA,
    ],
)


def _prep1_body(x_ref, w1_ref, dega_ref, degb_ref, y1_ref, dinv_ref):
  deg = dega_ref[...] + degb_ref[...] + 1.0
  dinv = lax.rsqrt(deg)
  xw = jnp.dot(x_ref[...], w1_ref[...], preferred_element_type=jnp.float32)
  dinv_ref[...] = dinv
  y1_ref[...] = xw * dinv


def _mid_body(a_ref, b_ref, y_ref, dinv_ref, b1_ref, g1_ref, bt1_ref,
              w2_ref, y2_ref):
  dinv = dinv_ref[...]
  pre = dinv * (a_ref[...] + b_ref[...] + y_ref[...]) + b1_ref[...]
  mu = jnp.mean(pre, axis=0, keepdims=True)
  d = pre - mu
  var = jnp.mean(d * d, axis=0, keepdims=True)
  h = jnp.maximum(g1_ref[...] * d * lax.rsqrt(var + 1e-5) + bt1_ref[...], 0.0)
  y2_ref[...] = jnp.dot(h, w2_ref[...],
                        preferred_element_type=jnp.float32) * dinv


N_POOL = 10240
NPT = N_POOL // NW
GP = G + 8


def _h2_body(a_ref, b_ref, y_ref, dinv_ref, b2_ref, g2_ref, bt2_ref, h_ref):
  dinv = dinv_ref[...]
  pre = dinv * (a_ref[...] + b_ref[...] + y_ref[...]) + b2_ref[...]
  mu = jnp.mean(pre, axis=0, keepdims=True)
  d = pre - mu
  var = jnp.mean(d * d, axis=0, keepdims=True)
  h = jnp.maximum(g2_ref[...] * d * lax.rsqrt(var + 1e-5) + bt2_ref[...], 0.0)
  h_ref[pl.ds(0, N), :] = h
  h_ref[pl.ds(N, N_POOL - N), :] = jnp.zeros((N_POOL - N, H), jnp.float32)


def _pool_body(h_hbm, bt_hbm, out_hbm, tbl, rows_v, bidx_v):
  c = lax.axis_index("c")
  s = lax.axis_index("s")
  w = c * NS + s
  base = w * NPT
  neg = jnp.full((16,), -1.0, jnp.float32)

  def zbody(i, carry):
    for k in range(8):
      tbl[i, pl.ds(k * 16, 16)] = neg
    return carry

  lax.fori_loop(0, GP, zbody, 0, unroll=False)

  def blkloop(blk, carry):
    pltpu.sync_copy(h_hbm.at[pl.ds(base + blk * 64, 64)], rows_v)
    pltpu.sync_copy(bt_hbm.at[pl.ds(base + blk * 64, 64)], bidx_v)

    def rowloop(g2, carry2):
      btv = bidx_v[pl.ds(g2 * 16, 16)]
      for k in range(16):
        bt = lax.squeeze(lax.slice(btv, (k,), (k + 1,)), (0,))
        r = g2 * 16 + k
        for kk in range(8):
          seg = rows_v[r, pl.ds(kk * 16, 16)]
          cur = tbl[bt, pl.ds(kk * 16, 16)]
          tbl[bt, pl.ds(kk * 16, 16)] = jnp.maximum(cur, seg)
      return carry2

    lax.fori_loop(0, 4, rowloop, 0, unroll=False)
    return carry

  lax.fori_loop(0, NPT // 64, blkloop, 0, unroll=False)
  pltpu.sync_copy(tbl, out_hbm.at[w])


_pool_call = pl.kernel(
    _pool_body,
    out_type=jax.ShapeDtypeStruct((NW, GP, H), jnp.float32),
    mesh=_MESH,
    scratch_types=[
        pltpu.VMEM((GP, H), jnp.float32),
        pltpu.VMEM((64, H), jnp.float32),
        pltpu.VMEM((64,), jnp.int32),
    ],
)


def _head_body(pool_ref, wl_ref, bl_ref, out_ref):
  m = jnp.max(pool_ref[...], axis=0)
  pooled = lax.slice(m, (0, 0), (G, H))
  pooled = jnp.where(pooled >= 0.0, pooled, 0.0)
  logits = jnp.dot(pooled, wl_ref[...], preferred_element_type=jnp.float32)
  out_ref[...] = 1.0 / (1.0 + jnp.exp(-(logits + bl_ref[...])))


def _tc_call(body, out_shapes, n_in, scratch_shapes=()):
  return pl.pallas_call(
      body,
      out_shape=out_shapes,
      in_specs=[pl.BlockSpec(memory_space=pltpu.VMEM) for _ in range(n_in)],
      out_specs=[pl.BlockSpec(memory_space=pltpu.VMEM) for _ in out_shapes],
      scratch_shapes=list(scratch_shapes),
  )


@jax.jit
def kernel(x, edge_index, batch, W1, b1, g1, beta1, W2, b2, g2, beta2, Wl, bl):
  src = edge_index[0]
  dst = edge_index[1]
  pad = E_PAD - E
  src_p = jnp.concatenate([src, jnp.zeros((pad,), jnp.int32)]).reshape(TOT_CHUNKS, CHUNK)
  dst_p = jnp.concatenate([dst, jnp.full((pad,), N, jnp.int32)]).reshape(TOT_CHUNKS, CHUNK)

  z128 = jnp.zeros((RPT, H), jnp.float32)

  degp = _deg_call(jnp.ones((CHUNK, H), jnp.float32), dst_p, z128)
  dega = degp[0, :N, 0:1]
  degb = degp[1, :N, 0:1]

  x_pad = jnp.pad(x, ((0, 0), (0, 16 - F_IN)))
  w1_pad = jnp.pad(W1, ((0, 16 - F_IN), (0, 0)))

  y1, dinv = _tc_call(
      _prep1_body,
      [jax.ShapeDtypeStruct((N, H), jnp.float32),
       jax.ShapeDtypeStruct((N, 1), jnp.float32)],
      4)(x_pad, w1_pad, dega, degb)

  agg1 = _agg128(y1, src_p, dst_p, z128)

  y2 = _tc_call(
      _mid_body,
      [jax.ShapeDtypeStruct((N, H), jnp.float32)],
      8)(agg1[0, :N], agg1[1, :N], y1, dinv,
         b1.reshape(1, H), g1.reshape(1, H), beta1.reshape(1, H), W2)[0]

  agg2 = _agg128(y2, src_p, dst_p, z128)

  h_pad = _tc_call(
      _h2_body,
      [jax.ShapeDtypeStruct((N_POOL, H), jnp.float32)],
      7)(agg2[0, :N], agg2[1, :N], y2, dinv,
         b2.reshape(1, H), g2.reshape(1, H), beta2.reshape(1, H))[0]

  batch_pad = jnp.concatenate([batch, jnp.full((N_POOL - N,), G, jnp.int32)])
  pools = _pool_call(h_pad, batch_pad)

  out = _tc_call(
      _head_body,
      [jax.ShapeDtypeStruct((G, 1), jnp.float32)],
      3)(pools, Wl, bl.reshape(1, 1))[0]
  return out


# speedup vs baseline: 20.5307x; 1.9719x over previous
import jax
import jax.numpy as jnp
from jax import lax
from jax.experimental import pallas as pl
from jax.experimental.pallas import tpu as pltpu
from jax.experimental.pallas import tpu_sc as plsc

N = 10000
E = 640000
F_IN = 9
H = 128
G = 128

NC = 2
NS = 16
NW = NC * NS
CHUNK = 120
KB = 16
K0 = 288
K1 = 48
KO0 = K0 // KB
KO1 = K1 // KB
TOT_CHUNKS = NS * (K0 + K1)
E_PAD = TOT_CHUNKS * CHUNK
KB_DEG = 8
K_DEG = TOT_CHUNKS // NW
KO_DEG = K_DEG // KB_DEG
RPT = 632
N_ACC = NS * RPT

_MESH = plsc.VectorSubcoreMesh(
    core_axis_name="c", subcore_axis_name="s", num_cores=NC, num_subcores=NS)


def _make_agg(W):
  def body(y_hbm, src_hbm, dst_hbm, z_hbm, out_hbm,
           acc, idx_s, idx_d, rows0, rows1, rows2,
           sg0, sg1, sg2, ss0, ss1, ss2):
    c = lax.axis_index("c")
    s = lax.axis_index("s")
    base = jnp.where(c == 0, s * K0, NS * K0 + s * K1)
    n_blocks = jnp.where(c == 0, KO0, KO1)
    rows = [rows0, rows1, rows2]
    sg = [sg0, sg1, sg2]
    ss = [ss0, ss1, ss2]
    pltpu.sync_copy(z_hbm, acc.at[pl.ds(s * RPT, RPT)])
    plsc.subcore_barrier()

    def body_outer(ko, carry):
      off = base + ko * KB
      pltpu.sync_copy(src_hbm.at[pl.ds(off, KB)], idx_s)
      pltpu.sync_copy(dst_hbm.at[pl.ds(off, KB)], idx_d)
      g = [None] * KB
      sc = [None] * KB
      g[0] = pltpu.async_copy(y_hbm.at[idx_s.at[0]], rows[0], sg[0])
      g[1] = pltpu.async_copy(y_hbm.at[idx_s.at[1]], rows[1], sg[1])
      for j in range(KB):
        b = j % 3
        g[j].wait()
        if j + 2 < KB:
          if j >= 1:
            sc[j - 1].wait()
          g[j + 2] = pltpu.async_copy(
              y_hbm.at[idx_s.at[j + 2]], rows[(j + 2) % 3], sg[(j + 2) % 3])
        sc[j] = pltpu.async_copy(rows[b], acc.at[idx_d.at[j]], ss[b],
                                 add=True)
      sc[KB - 3].wait()
      sc[KB - 2].wait()
      sc[KB - 1].wait()
      return carry

    lax.fori_loop(0, n_blocks, body_outer, 0, unroll=False)
    plsc.subcore_barrier()
    pltpu.sync_copy(acc.at[pl.ds(s * RPT, RPT)],
                    out_hbm.at[c, pl.ds(s * RPT, RPT)])

  return pl.kernel(
      body,
      out_type=jax.ShapeDtypeStruct((NC, N_ACC, W), jnp.float32),
      mesh=_MESH,
      scratch_types=[
          pltpu.VMEM_SHARED((N_ACC, W), jnp.float32),
          pltpu.VMEM((KB, CHUNK), jnp.int32),
          pltpu.VMEM((KB, CHUNK), jnp.int32),
          pltpu.VMEM((CHUNK, W), jnp.float32),
          pltpu.VMEM((CHUNK, W), jnp.float32),
          pltpu.VMEM((CHUNK, W), jnp.float32),
          pltpu.SemaphoreType.DMA,
          pltpu.SemaphoreType.DMA,
          pltpu.SemaphoreType.DMA,
          pltpu.SemaphoreType.DMA,
          pltpu.SemaphoreType.DMA,
          pltpu.SemaphoreType.DMA,
      ],
  )


_agg128 = _make_agg(H)


def _degc_body(ones_hbm, dst_hbm, z_hbm, out_hbm, acc, idx_d, rows_v,
               ss0, ss1):
  c = lax.axis_index("c")
  s = lax.axis_index("s")
  w = c * NS + s
  ss = [ss0, ss1]
  pltpu.sync_copy(ones_hbm, rows_v)
  pltpu.sync_copy(z_hbm, acc.at[pl.ds(s * RPT, RPT)])
  plsc.subcore_barrier()

  def body_outer(ko, carry):
    pltpu.sync_copy(dst_hbm.at[pl.ds(w * K_DEG + ko * KB_DEG, KB_DEG)], idx_d)
    sc = [None] * KB_DEG
    for j in range(KB_DEG):
      b = j % 2
      if j >= 2:
        sc[j - 2].wait()
      sc[j] = pltpu.async_copy(rows_v, acc.at[idx_d.at[j]], ss[b], add=True)
    sc[KB_DEG - 2].wait()
    sc[KB_DEG - 1].wait()
    return carry

  lax.fori_loop(0, KO_DEG, body_outer, 0, unroll=False)
  plsc.subcore_barrier()
  pltpu.sync_copy(acc.at[pl.ds(s * RPT, RPT)],
                  out_hbm.at[c, pl.ds(s * RPT, RPT)])


_deg_call = pl.kernel(
    _degc_body,
    out_type=jax.ShapeDtypeStruct((NC, N_ACC, H), jnp.float32),
    mesh=_MESH,
    scratch_types=[
        pltpu.VMEM_SHARED((N_ACC, H), jnp.float32),
        pltpu.VMEM((KB_DEG, CHUNK), jnp.int32),
        pltpu.VMEM((CHUNK, H), jnp.float32),
        pltpu.SemaphoreType.DMA,
        pltpu.SemaphoreType.DMA,
    ],
)


def _prep1_body(x_ref, w1_ref, dega_ref, degb_ref, y1_ref, dinv_ref):
  deg = dega_ref[...] + degb_ref[...] + 1.0
  dinv = lax.rsqrt(deg)
  xw = jnp.dot(x_ref[...], w1_ref[...], preferred_element_type=jnp.float32)
  dinv_ref[...] = dinv
  y1_ref[...] = xw * dinv


def _mid_body(a_ref, b_ref, y_ref, dinv_ref, b1_ref, g1_ref, bt1_ref,
              w2_ref, y2_ref):
  dinv = dinv_ref[...]
  pre = dinv * (a_ref[...] + b_ref[...] + y_ref[...]) + b1_ref[...]
  mu = jnp.mean(pre, axis=0, keepdims=True)
  d = pre - mu
  var = jnp.mean(d * d, axis=0, keepdims=True)
  h = jnp.maximum(g1_ref[...] * d * lax.rsqrt(var + 1e-5) + bt1_ref[...], 0.0)
  y2_ref[...] = jnp.dot(h, w2_ref[...],
                        preferred_element_type=jnp.float32) * dinv


N_POOL = 10240
NPT = N_POOL // NW
GP = G + 8


def _h2_body(a_ref, b_ref, y_ref, dinv_ref, b2_ref, g2_ref, bt2_ref, h_ref):
  dinv = dinv_ref[...]
  pre = dinv * (a_ref[...] + b_ref[...] + y_ref[...]) + b2_ref[...]
  mu = jnp.mean(pre, axis=0, keepdims=True)
  d = pre - mu
  var = jnp.mean(d * d, axis=0, keepdims=True)
  h = jnp.maximum(g2_ref[...] * d * lax.rsqrt(var + 1e-5) + bt2_ref[...], 0.0)
  h_ref[pl.ds(0, N), :] = h
  h_ref[pl.ds(N, N_POOL - N), :] = jnp.zeros((N_POOL - N, H), jnp.float32)


def _pool_body(h_hbm, bt_hbm, out_hbm, tbl, rows_v, bidx_v):
  c = lax.axis_index("c")
  s = lax.axis_index("s")
  w = c * NS + s
  base = w * NPT
  neg = jnp.full((16,), -1.0, jnp.float32)

  def zbody(i, carry):
    for k in range(8):
      tbl[i, pl.ds(k * 16, 16)] = neg
    return carry

  lax.fori_loop(0, GP, zbody, 0, unroll=False)

  def blkloop(blk, carry):
    pltpu.sync_copy(h_hbm.at[pl.ds(base + blk * 64, 64)], rows_v)
    pltpu.sync_copy(bt_hbm.at[pl.ds(base + blk * 64, 64)], bidx_v)

    def rowloop(g2, carry2):
      btv = bidx_v[pl.ds(g2 * 16, 16)]
      for k in range(16):
        bt = lax.squeeze(lax.slice(btv, (k,), (k + 1,)), (0,))
        r = g2 * 16 + k
        for kk in range(8):
          seg = rows_v[r, pl.ds(kk * 16, 16)]
          cur = tbl[bt, pl.ds(kk * 16, 16)]
          tbl[bt, pl.ds(kk * 16, 16)] = jnp.maximum(cur, seg)
      return carry2

    lax.fori_loop(0, 4, rowloop, 0, unroll=False)
    return carry

  lax.fori_loop(0, NPT // 64, blkloop, 0, unroll=False)
  pltpu.sync_copy(tbl, out_hbm.at[w])


_pool_call = pl.kernel(
    _pool_body,
    out_type=jax.ShapeDtypeStruct((NW, GP, H), jnp.float32),
    mesh=_MESH,
    scratch_types=[
        pltpu.VMEM((GP, H), jnp.float32),
        pltpu.VMEM((64, H), jnp.float32),
        pltpu.VMEM((64,), jnp.int32),
    ],
)


def _head_body(pool_ref, wl_ref, bl_ref, out_ref):
  m = jnp.max(pool_ref[...], axis=0)
  pooled = lax.slice(m, (0, 0), (G, H))
  pooled = jnp.where(pooled >= 0.0, pooled, 0.0)
  logits = jnp.dot(pooled, wl_ref[...], preferred_element_type=jnp.float32)
  out_ref[...] = 1.0 / (1.0 + jnp.exp(-(logits + bl_ref[...])))


def _tc_call(body, out_shapes, n_in, scratch_shapes=()):
  return pl.pallas_call(
      body,
      out_shape=out_shapes,
      in_specs=[pl.BlockSpec(memory_space=pltpu.VMEM) for _ in range(n_in)],
      out_specs=[pl.BlockSpec(memory_space=pltpu.VMEM) for _ in out_shapes],
      scratch_shapes=list(scratch_shapes),
  )


@jax.jit
def kernel(x, edge_index, batch, W1, b1, g1, beta1, W2, b2, g2, beta2, Wl, bl):
  src = edge_index[0]
  dst = edge_index[1]
  pad = E_PAD - E
  src_p = jnp.concatenate([src, jnp.zeros((pad,), jnp.int32)]).reshape(TOT_CHUNKS, CHUNK)
  dst_p = jnp.concatenate([dst, jnp.full((pad,), N, jnp.int32)]).reshape(TOT_CHUNKS, CHUNK)

  z128 = jnp.zeros((RPT, H), jnp.float32)

  degp = _deg_call(jnp.ones((CHUNK, H), jnp.float32), dst_p, z128)
  dega = degp[0, :N, 0:1]
  degb = degp[1, :N, 0:1]

  x_pad = jnp.pad(x, ((0, 0), (0, 16 - F_IN)))
  w1_pad = jnp.pad(W1, ((0, 16 - F_IN), (0, 0)))

  y1, dinv = _tc_call(
      _prep1_body,
      [jax.ShapeDtypeStruct((N, H), jnp.float32),
       jax.ShapeDtypeStruct((N, 1), jnp.float32)],
      4)(x_pad, w1_pad, dega, degb)

  agg1 = _agg128(y1, src_p, dst_p, z128)

  y2 = _tc_call(
      _mid_body,
      [jax.ShapeDtypeStruct((N, H), jnp.float32)],
      8)(agg1[0, :N], agg1[1, :N], y1, dinv,
         b1.reshape(1, H), g1.reshape(1, H), beta1.reshape(1, H), W2)[0]

  agg2 = _agg128(y2, src_p, dst_p, z128)

  h_pad = _tc_call(
      _h2_body,
      [jax.ShapeDtypeStruct((N_POOL, H), jnp.float32)],
      7)(agg2[0, :N], agg2[1, :N], y2, dinv,
         b2.reshape(1, H), g2.reshape(1, H), beta2.reshape(1, H))[0]

  batch_pad = jnp.concatenate([batch, jnp.full((N_POOL - N,), G, jnp.int32)])
  pools = _pool_call(h_pad, batch_pad)

  out = _tc_call(
      _head_body,
      [jax.ShapeDtypeStruct((G, 1), jnp.float32)],
      3)(pools, Wl, bl.reshape(1, 1))[0]
  return out
